# Initial kernel scaffold; baseline (speedup 1.0000x reference)
import jax
import jax.numpy as jnp
from jax.experimental import pallas as pl


def kernel(tokens, table):
    raise NotImplementedError("write your pallas kernel here")


# speedup vs baseline: 1.0197x; 1.0197x over previous
import functools
import math

import jax
import jax.numpy as jnp
from jax import lax
from jax.experimental import pallas as pl
from jax.experimental.pallas import tpu as pltpu
from jax.experimental.pallas import tpu_sc as plsc

NC = 2
NS = 16
NW = NC * NS
LANES = 16


def _emb_kernel(B, D, C):
    n_chunks_w = (B // NW) // C
    b_per_w = B // NW
    scale = math.sqrt(D)
    mesh = plsc.VectorSubcoreMesh(
        core_axis_name="c", subcore_axis_name="s", num_cores=NC, num_subcores=NS
    )

    @functools.partial(
        pl.kernel,
        mesh=mesh,
        compiler_params=pltpu.CompilerParams(use_tc_tiling_on_sc=False),
        out_type=jax.ShapeDtypeStruct((B, D), jnp.float32),
        scratch_types=[
            pltpu.VMEM((C,), jnp.int32),
            pltpu.VMEM((C, D), jnp.float32),
            pltpu.SemaphoreType.DMA,
        ],
    )
    def k(tok_hbm, tab_hbm, out_hbm, idx_v, rows_v, sem):
        wid = lax.axis_index("s") * NC + lax.axis_index("c")
        base = wid * b_per_w

        def chunk_body(ci, carry):
            off = base + ci * C
            pltpu.sync_copy(tok_hbm.at[pl.ds(off, C)], idx_v)
            pltpu.async_copy(tab_hbm.at[idx_v], rows_v, sem).wait()

            def srow(r, carry2):
                r0 = r * 8
                for j in range(8):
                    rows_v[r0 + j, pl.ds(0, LANES)] = (
                        rows_v[r0 + j, pl.ds(0, LANES)] * scale
                    )
                    rows_v[r0 + j, pl.ds(LANES, LANES)] = (
                        rows_v[r0 + j, pl.ds(LANES, LANES)] * scale
                    )
                return carry2

            lax.fori_loop(0, C // 8, srow, 0)
            pltpu.sync_copy(rows_v, out_hbm.at[pl.ds(off, C)])
            return carry

        lax.fori_loop(0, n_chunks_w, chunk_body, 0)

    return k


def kernel(tokens, table):
    B = tokens.shape[0] * tokens.shape[1]
    D = table.shape[1]
    toks = tokens.reshape(B).astype(jnp.int32)
    out = _emb_kernel(B, D, 1280)(toks, table)
    return out.reshape(tokens.shape + (D,))


# speedup vs baseline: 2.1467x; 2.1052x over previous
import functools
import math

import jax
import jax.numpy as jnp
from jax import lax
from jax.experimental import pallas as pl
from jax.experimental.pallas import tpu as pltpu
from jax.experimental.pallas import tpu_sc as plsc

NC = 2
NS = 16
NW = NC * NS
LANES = 16

V = 1000000
EMB = 32
NSEQ = 50
NB = 16384
SCALE = math.sqrt(float(EMB))

TC_BLK = 4096
TC_NBLK = (V + TC_BLK - 1) // TC_BLK
TP = TC_NBLK * (TC_BLK // 4)


def _tc_transform_body(tt_ref, out_ref):
    x = tt_ref[...]
    xT = jnp.transpose(x)
    x3 = xT.reshape(TC_BLK // 4, 4, EMB)
    y = jnp.concatenate([x3[:, q, :] for q in range(4)], axis=1)
    out_ref[...] = y * SCALE


def _tc_transform(tt):
    return pl.pallas_call(
        _tc_transform_body,
        grid=(TC_NBLK,),
        in_specs=[pl.BlockSpec((EMB, TC_BLK), lambda j: (0, j))],
        out_specs=pl.BlockSpec((TC_BLK // 4, 128), lambda j: (j, 0)),
        out_shape=jax.ShapeDtypeStruct((TP, 128), jnp.float32),
    )(tt)


N_UNITS = NSEQ * (NB // 128)
U_PER_W = N_UNITS // NW


def _sc_gather():
    mesh = plsc.VectorSubcoreMesh(
        core_axis_name="c", subcore_axis_name="s", num_cores=NC, num_subcores=NS
    )

    @functools.partial(
        pl.kernel,
        mesh=mesh,
        compiler_params=pltpu.CompilerParams(
            use_tc_tiling_on_sc=False, needs_layout_passes=False
        ),
        out_type=jax.ShapeDtypeStruct((NSEQ, 4, NB // 128, 8, 128), jnp.float32),
        scratch_types=[
            pltpu.VMEM((U_PER_W * 128,), jnp.int32),
            [pltpu.VMEM((256,), jnp.int32) for _ in range(4)],
            [pltpu.VMEM((256, 16), jnp.float32) for _ in range(4)],
            [pltpu.VMEM((4, 8, 128), jnp.float32) for _ in range(4)],
            [pltpu.SemaphoreType.DMA for _ in range(4)],
            [pltpu.SemaphoreType.DMA for _ in range(4)],
        ],
    )
    def k(tok_hbm, tab_hbm, out_hbm, tok_v, idx, rows, obuf, sem, osem):
        wid = lax.axis_index("s") * NC + lax.axis_index("c")
        base = wid * U_PER_W
        last = base + U_PER_W - 1
        iota = lax.iota(jnp.int32, 16)

        pltpu.sync_copy(tok_hbm.at[pl.ds(base * 128, U_PER_W * 128)], tok_v)

        def build_idx(u, idx_v):
            lo = (u - base) * 128
            for g in range(8):
                t2 = tok_v[pl.ds(lo + g * 16, 16)] * 2
                idx_v[pl.ds(g * 16, 16)] = t2
                idx_v[pl.ds(128 + g * 16, 16)] = t2 + 1

        def drain(u, rows_v, ob, osm):
            s = u // (NB // 128)
            bh = u % (NB // 128)

            def body_e(e, carry):
                h128 = (e // 16) * 128
                csp = iota * 0 + (e % 16)
                for g in range(8):
                    v = plsc.load_gather(
                        rows_v, [iota + (g * 16) + h128, csp]
                    )
                    ob[e // 8, e % 8, pl.ds(g * 16, 16)] = v
                return carry

            lax.fori_loop(0, EMB, body_e, 0, unroll=2)
            for ehi in range(4):
                pltpu.async_copy(ob.at[ehi], out_hbm.at[s, ehi, bh], osm)

        def owait(ob, osm):
            for ehi in range(4):
                pltpu.make_async_copy(
                    ob.at[ehi], out_hbm.at[0, ehi, 0], osm
                ).wait()

        for b in range(3):
            build_idx(base + b, idx[b])
            pltpu.async_copy(tab_hbm.at[idx[b]], rows[b], sem[b])

        def quad(p, carry):
            u0 = base + 4 * p
            for b in range(4):
                u = u0 + b
                nb = (b + 3) % 4
                build_idx(jnp.minimum(u + 3, last), idx[nb])
                pltpu.async_copy(tab_hbm.at[idx[nb]], rows[nb], sem[nb])
                pltpu.make_async_copy(tab_hbm.at[idx[b]], rows[b], sem[b]).wait()

                @pl.when(p > 0)
                def _():
                    owait(obuf[b], osem[b])

                drain(u, rows[b], obuf[b], osem[b])
            return carry

        lax.fori_loop(0, U_PER_W // 4, quad, 0)
        for b in range(4):
            owait(obuf[b], osem[b])
        for b in range(3):
            pltpu.make_async_copy(tab_hbm.at[idx[b]], rows[b], sem[b]).wait()

    return k


def kernel(tokens, table):
    tt = table.T
    t128 = _tc_transform(tt)
    tab16 = t128.reshape(TP * 8, 16)
    toks = tokens.T.reshape(NB * NSEQ).astype(jnp.int32)
    out5 = _sc_gather()(toks, tab16)
    return out5.transpose(2, 4, 0, 1, 3).reshape(NB, NSEQ, EMB)
